# baseline (speedup 1.0000x reference)
import jax
import jax.numpy as jnp
from jax.experimental import pallas as pl


def _linear_kernel(x_ref, w_ref, b_ref, o_ref):
    n = pl.program_id(1)
    BN = o_ref.shape[1]
    w = w_ref[pl.ds(n * BN, BN), :]
    bias = b_ref[:, pl.ds(n * BN, BN)]
    o_ref[...] = jax.lax.dot_general(
        x_ref[...].astype(jnp.bfloat16), w.astype(jnp.bfloat16),
        dimension_numbers=(((1,), (1,)), ((), ())),
        preferred_element_type=jnp.float32,
    ) + bias


def kernel(values, sequence_ids, W, b):
    del sequence_ids
    B, S, K = values.shape
    N = W.shape[0]
    M = B * S
    x = values.reshape(M, K)
    BM = 2048
    BN = 512
    out = pl.pallas_call(
        _linear_kernel,
        grid=(M // BM, N // BN),
        in_specs=[
            pl.BlockSpec((BM, K), lambda i, n: (i, 0)),
            pl.BlockSpec((N, K), lambda i, n: (0, 0)),
            pl.BlockSpec((1, N), lambda i, n: (0, 0)),
        ],
        out_specs=pl.BlockSpec((BM, BN), lambda i, n: (i, n)),
        out_shape=jax.ShapeDtypeStruct((M, N), jnp.float32),
    )(x, W, b.reshape(1, N))
    return out.reshape(B, S, N)
